# Initial kernel scaffold; baseline (speedup 1.0000x reference)
import jax
import jax.numpy as jnp
from jax.experimental import pallas as pl


def kernel(x, W_gate, g1, bn1, g2, bn2, Wf1, bf1, Wf2, bf2, W_in, conv_w, conv_b, W_xproj, W_dt, b_dt, A_log, Dp, W_out):
    raise NotImplementedError("write your pallas kernel here")


# speedup vs baseline: 7.7836x; 7.7836x over previous
import functools
import math

import jax
import jax.numpy as jnp
from jax import lax
from jax.experimental import pallas as pl
from jax.experimental.pallas import tpu as pltpu

F32 = jnp.float32


def _silu(v):
    return v * jax.nn.sigmoid(v)


def _router_body(x_ref, g1_ref, b1_ref, wg_ref, xn_ref, w_ref):
    x = x_ref[...]
    m = jnp.mean(x, axis=1, keepdims=True)
    xc = x - m
    v = jnp.mean(xc * xc, axis=1, keepdims=True)
    xn = xc * lax.rsqrt(v + 1e-5) * g1_ref[...] + b1_ref[...]
    xn_ref[...] = xn
    logits = jnp.dot(xn, wg_ref[...], preferred_element_type=F32)
    C, E = logits.shape
    lmax = jnp.max(logits, axis=1, keepdims=True)
    p = jnp.exp(logits - lmax)
    col = lax.broadcasted_iota(jnp.int32, (C, E), 1)
    m1 = jnp.max(p, axis=1, keepdims=True)
    found = jnp.zeros((C, 1), dtype=jnp.bool_)
    sel = jnp.zeros((C, E), dtype=jnp.bool_)
    for e in range(E):
        s_e = jnp.sum(jnp.where(col == e, p, 0.0), axis=1, keepdims=True)
        take = jnp.logical_and(s_e == m1, jnp.logical_not(found))
        found = jnp.logical_or(found, take)
        sel = jnp.logical_or(sel, jnp.logical_and(take, col == e))
    p2 = jnp.where(sel, -jnp.inf, p)
    m2 = jnp.max(p2, axis=1, keepdims=True)
    found2 = jnp.zeros((C, 1), dtype=jnp.bool_)
    for e in range(E):
        s_e = jnp.sum(jnp.where(col == e, p2, -jnp.inf), axis=1, keepdims=True)
        take = jnp.logical_and(s_e == m2, jnp.logical_not(found2))
        found2 = jnp.logical_or(found2, take)
        sel = jnp.logical_or(sel, jnp.logical_and(take, col == e))
    denom = jnp.clip(m1 + m2, 1e-9, None)
    w_ref[...] = jnp.where(sel, p, 0.0) / denom


def _mamba_body(xn_ref, w_ref, win_ref, cw_ref, cb_ref, wxp_ref, wdt_ref,
                bdt_ref, at_ref, dp_ref, wout_ref, out_ref,
                xb_ref, h_ref, ys_ref, dl_ref, uc_ref, bc_ref,
                *, C, d_inner, d_state, dt_rank, K):
    e = pl.program_id(0)
    c = pl.program_id(1)
    xz = jnp.dot(xn_ref[...], win_ref[0], preferred_element_type=F32)
    x_ = xz[:, :d_inner]
    z = xz[:, d_inner:]

    @pl.when(c == 0)
    def _():
        xb_ref[0:K - 1, :] = jnp.zeros((K - 1, d_inner), F32)

    @pl.when(c != 0)
    def _():
        xb_ref[0:K - 1, :] = xb_ref[C:C + K - 1, :]

    xb_ref[K - 1:C + K - 1, :] = x_
    xc = jnp.zeros((C, d_inner), F32)
    for k in range(K):
        xc = xc + xb_ref[k:k + C, :] * cw_ref[0, k:k + 1, :]
    xc = _silu(xc + cb_ref[0])

    x_dbl = jnp.dot(xc, wxp_ref[0], preferred_element_type=F32)
    dt = x_dbl[:, :dt_rank]
    bc_ref[...] = x_dbl[:, dt_rank:dt_rank + 2 * d_state]
    dl_ref[...] = jax.nn.softplus(
        jnp.dot(dt, wdt_ref[0], preferred_element_type=F32) + bdt_ref[0])
    uc_ref[...] = xc
    A = -jnp.exp(at_ref[0])
    r_i = lax.broadcasted_iota(jnp.int32, (d_state, 2 * d_state), 0)
    c_i = lax.broadcasted_iota(jnp.int32, (d_state, 2 * d_state), 1)
    mask_b = jnp.where(c_i == r_i, 1.0, 0.0).astype(F32)
    mask_c = jnp.where(c_i == r_i + d_state, 1.0, 0.0).astype(F32)

    @pl.when(c == 0)
    def _():
        h_ref[...] = jnp.zeros((d_state, d_inner), F32)

    def step(t, h):
        d_t = dl_ref[pl.ds(t, 1), :]
        u_t = uc_ref[pl.ds(t, 1), :]
        bc_row = bc_ref[pl.ds(t, 1), :]
        b_t = jnp.sum(bc_row * mask_b, axis=1, keepdims=True)
        c_t = jnp.sum(bc_row * mask_c, axis=1, keepdims=True)
        dA = jnp.exp(d_t * A)
        h = dA * h + (d_t * u_t) * b_t
        ys_ref[pl.ds(t, 1), :] = jnp.sum(h * c_t, axis=0, keepdims=True)
        return h

    h_ref[...] = lax.fori_loop(0, C, step, h_ref[...])

    yf = (ys_ref[...] + uc_ref[...] * dp_ref[0]) * _silu(z)
    col = lax.broadcasted_iota(jnp.int32, w_ref.shape, 1)
    wcol = jnp.sum(jnp.where(col == e, w_ref[...], 0.0), axis=1,
                   keepdims=True)
    contrib = jnp.dot(yf * wcol, wout_ref[0], preferred_element_type=F32)
    rows = pl.ds(c * C, C)

    @pl.when(e == 0)
    def _():
        out_ref[rows, :] = contrib

    @pl.when(e != 0)
    def _():
        out_ref[rows, :] = out_ref[rows, :] + contrib


def _ffn_body(x_ref, f_ref, g2_ref, b2_ref, wf1_ref, bf1_ref, wf2_ref,
              bf2_ref, o_ref):
    x2 = x_ref[...] + f_ref[...]
    m = jnp.mean(x2, axis=1, keepdims=True)
    xc = x2 - m
    v = jnp.mean(xc * xc, axis=1, keepdims=True)
    xn = xc * lax.rsqrt(v + 1e-5) * g2_ref[...] + b2_ref[...]
    h = jnp.dot(xn, wf1_ref[...], preferred_element_type=F32) + bf1_ref[...]
    h = 0.5 * h * (1.0 + lax.erf(h / jnp.sqrt(2.0).astype(F32)))
    o_ref[...] = x2 + jnp.dot(h, wf2_ref[...],
                              preferred_element_type=F32) + bf2_ref[...]


def kernel(x, W_gate, g1, bn1, g2, bn2, Wf1, bf1, Wf2, bf2,
           W_in, conv_w, conv_b, W_xproj, W_dt, b_dt, A_log, Dp, W_out):
    B, L, D = x.shape
    E = W_gate.shape[1]
    d_inner = conv_w.shape[1]
    K = conv_w.shape[2]
    d_state = A_log.shape[2]
    dt_rank = W_dt.shape[1]
    C = 256
    NC = L // C
    x2d = x.reshape(L, D)
    cwT = jnp.swapaxes(conv_w, 1, 2)
    AT = jnp.swapaxes(A_log, 1, 2)
    row = lambda a: a.reshape(1, -1)

    xn, w = pl.pallas_call(
        _router_body,
        grid=(NC,),
        in_specs=[
            pl.BlockSpec((C, D), lambda c: (c, 0)),
            pl.BlockSpec((1, D), lambda c: (0, 0)),
            pl.BlockSpec((1, D), lambda c: (0, 0)),
            pl.BlockSpec((D, E), lambda c: (0, 0)),
        ],
        out_specs=[
            pl.BlockSpec((C, D), lambda c: (c, 0)),
            pl.BlockSpec((C, E), lambda c: (c, 0)),
        ],
        out_shape=[
            jax.ShapeDtypeStruct((L, D), F32),
            jax.ShapeDtypeStruct((L, E), F32),
        ],
    )(x2d, row(g1), row(bn1), W_gate)

    body = functools.partial(_mamba_body, C=C, d_inner=d_inner,
                             d_state=d_state, dt_rank=dt_rank, K=K)
    fused = pl.pallas_call(
        body,
        grid=(E, NC),
        in_specs=[
            pl.BlockSpec((C, D), lambda e, c: (c, 0)),
            pl.BlockSpec((C, E), lambda e, c: (c, 0)),
            pl.BlockSpec((1, D, 2 * d_inner), lambda e, c: (e, 0, 0)),
            pl.BlockSpec((1, K, d_inner), lambda e, c: (e, 0, 0)),
            pl.BlockSpec((1, 1, d_inner), lambda e, c: (e, 0, 0)),
            pl.BlockSpec((1, d_inner, dt_rank + 2 * d_state),
                         lambda e, c: (e, 0, 0)),
            pl.BlockSpec((1, dt_rank, d_inner), lambda e, c: (e, 0, 0)),
            pl.BlockSpec((1, 1, d_inner), lambda e, c: (e, 0, 0)),
            pl.BlockSpec((1, d_state, d_inner), lambda e, c: (e, 0, 0)),
            pl.BlockSpec((1, 1, d_inner), lambda e, c: (e, 0, 0)),
            pl.BlockSpec((1, d_inner, D), lambda e, c: (e, 0, 0)),
        ],
        out_specs=pl.BlockSpec((L, D), lambda e, c: (0, 0)),
        out_shape=jax.ShapeDtypeStruct((L, D), F32),
        scratch_shapes=[
            pltpu.VMEM((C + 8, d_inner), F32),
            pltpu.VMEM((d_state, d_inner), F32),
            pltpu.VMEM((C, d_inner), F32),
            pltpu.VMEM((C, d_inner), F32),
            pltpu.VMEM((C, d_inner), F32),
            pltpu.VMEM((C, 2 * d_state), F32),
        ],
        compiler_params=pltpu.CompilerParams(
            dimension_semantics=("arbitrary", "arbitrary")),
    )(xn, w, W_in, cwT, conv_b[:, None], W_xproj, W_dt, b_dt[:, None],
      AT, Dp[:, None], W_out)

    out = pl.pallas_call(
        _ffn_body,
        grid=(NC,),
        in_specs=[
            pl.BlockSpec((C, D), lambda c: (c, 0)),
            pl.BlockSpec((C, D), lambda c: (c, 0)),
            pl.BlockSpec((1, D), lambda c: (0, 0)),
            pl.BlockSpec((1, D), lambda c: (0, 0)),
            pl.BlockSpec((D, 2 * D), lambda c: (0, 0)),
            pl.BlockSpec((1, 2 * D), lambda c: (0, 0)),
            pl.BlockSpec((2 * D, D), lambda c: (0, 0)),
            pl.BlockSpec((1, D), lambda c: (0, 0)),
        ],
        out_specs=pl.BlockSpec((C, D), lambda c: (c, 0)),
        out_shape=jax.ShapeDtypeStruct((L, D), F32),
    )(x2d, fused, row(g2), row(bn2), Wf1, row(bf1), Wf2, row(bf2))

    return out.reshape(B, L, D)


# speedup vs baseline: 13.9542x; 1.7928x over previous
import functools
import math

import jax
import jax.numpy as jnp
from jax import lax
from jax.experimental import pallas as pl
from jax.experimental.pallas import tpu as pltpu
from jax.experimental.pallas import tpu_sc as plsc

F32 = jnp.float32
_NEG = -3.0e38


def _silu(v):
    return v * jax.nn.sigmoid(v)


def _router_body(x_ref, g1_ref, b1_ref, wg_ref, xn_ref, p_ref):
    x = x_ref[...]
    m = jnp.mean(x, axis=1, keepdims=True)
    xc = x - m
    v = jnp.mean(xc * xc, axis=1, keepdims=True)
    xn = xc * lax.rsqrt(v + 1e-5) * g1_ref[...] + b1_ref[...]
    xn_ref[...] = xn
    logits = jnp.dot(xn, wg_ref[...], preferred_element_type=F32)
    lmax = jnp.max(logits, axis=1, keepdims=True)
    p_ref[...] = jnp.exp(logits - lmax)


def _make_sc_router(E, L):
    info = plsc.get_sparse_core_info()
    NC, NS = info.num_cores, info.num_subcores
    NW = NC * NS
    TOK = L // NW
    mesh = plsc.VectorSubcoreMesh(core_axis_name="c", subcore_axis_name="s")

    @functools.partial(
        pl.kernel, mesh=mesh,
        out_type=jax.ShapeDtypeStruct((E, L), F32),
        scratch_types=[
            pltpu.VMEM((E, TOK), F32),
            pltpu.VMEM((E, TOK), F32),
        ],
    )
    def router(p_hbm, w_hbm, scr, wv):
        wid = lax.axis_index("s") * NC + lax.axis_index("c")
        base = wid * TOK
        for e in range(E):
            pltpu.sync_copy(p_hbm.at[e, pl.ds(base, TOK)], scr.at[e])
        one = jnp.full((16,), 1.0, F32)
        zero = jnp.zeros((16,), F32)
        for j in range(TOK // 16):
            sl = pl.ds(16 * j, 16)
            rows = [scr[e, sl] for e in range(E)]
            m1 = rows[0]
            for e in range(1, E):
                m1 = jnp.maximum(m1, rows[e])
            found = zero
            sel = []
            for e in range(E):
                ge = jnp.where(rows[e] >= m1, one, zero)
                take = ge * (1.0 - found)
                found = found + take
                sel.append(take)
            masked = [rows[e] + sel[e] * _NEG for e in range(E)]
            m2 = masked[0]
            for e in range(1, E):
                m2 = jnp.maximum(m2, masked[e])
            found2 = zero
            for e in range(E):
                ge2 = jnp.where(masked[e] >= m2, one, zero)
                take2 = ge2 * (1.0 - found2)
                found2 = found2 + take2
                sel[e] = sel[e] + take2
            denom = jnp.maximum(m1 + m2, 1e-9)
            for e in range(E):
                wv[e, sl] = rows[e] * sel[e] / denom
        for e in range(E):
            pltpu.sync_copy(wv.at[e], w_hbm.at[e, pl.ds(base, TOK)])

    return router


def _mamba_body(xn_ref, w_ref, win_ref, cw_ref, cb_ref, wxp_ref, wdt_ref,
                bdt_ref, at_ref, dp_ref, wout_ref, out_ref,
                xb_ref, h_ref, ys_ref, dl_ref, uc_ref, du_ref, bc_ref,
                *, C, d_inner, d_state, dt_rank, K):
    e = pl.program_id(0)
    c = pl.program_id(1)
    xz = jnp.dot(xn_ref[...], win_ref[0], preferred_element_type=F32)
    x_ = xz[:, :d_inner]
    z = xz[:, d_inner:]

    @pl.when(c == 0)
    def _():
        xb_ref[0:K - 1, :] = jnp.zeros((K - 1, d_inner), F32)

    @pl.when(c != 0)
    def _():
        xb_ref[0:K - 1, :] = xb_ref[C:C + K - 1, :]

    xb_ref[K - 1:C + K - 1, :] = x_
    xc = jnp.zeros((C, d_inner), F32)
    for k in range(K):
        xc = xc + xb_ref[k:k + C, :] * cw_ref[0, k:k + 1, :]
    xc = _silu(xc + cb_ref[0])

    x_dbl = jnp.dot(xc, wxp_ref[0], preferred_element_type=F32)
    dt = x_dbl[:, :dt_rank]
    bc_ref[...] = x_dbl[:, dt_rank:dt_rank + 2 * d_state]
    dl_ref[...] = jax.nn.softplus(
        jnp.dot(dt, wdt_ref[0], preferred_element_type=F32) + bdt_ref[0])
    uc_ref[...] = xc
    du_ref[...] = dl_ref[...] * xc
    A = -jnp.exp(at_ref[0])
    r_i = lax.broadcasted_iota(jnp.int32, (d_state, 2 * d_state), 0)
    c_i = lax.broadcasted_iota(jnp.int32, (d_state, 2 * d_state), 1)
    mask_b = jnp.where(c_i == r_i, 1.0, 0.0).astype(F32)
    mask_c = jnp.where(c_i == r_i + d_state, 1.0, 0.0).astype(F32)

    @pl.when(c == 0)
    def _():
        h_ref[...] = jnp.zeros((d_state, d_inner), F32)

    def step(t, h):
        d_t = dl_ref[pl.ds(t, 1), :]
        du_t = du_ref[pl.ds(t, 1), :]
        bc_row = bc_ref[pl.ds(t, 1), :]
        b_t = jnp.sum(bc_row * mask_b, axis=1, keepdims=True)
        c_t = jnp.sum(bc_row * mask_c, axis=1, keepdims=True)
        dA = jnp.exp(d_t * A)
        h = dA * h + du_t * b_t
        ys_ref[pl.ds(t, 1), :] = jnp.sum(h * c_t, axis=0, keepdims=True)
        return h

    h_ref[...] = lax.fori_loop(0, C, step, h_ref[...], unroll=4)

    yf = (ys_ref[...] + uc_ref[...] * dp_ref[0]) * _silu(z)
    col = lax.broadcasted_iota(jnp.int32, w_ref.shape, 1)
    wcol = jnp.sum(jnp.where(col == e, w_ref[...], 0.0), axis=1,
                   keepdims=True)
    contrib = jnp.dot(yf * wcol, wout_ref[0], preferred_element_type=F32)
    rows = pl.ds(c * C, C)

    @pl.when(e == 0)
    def _():
        out_ref[rows, :] = contrib

    @pl.when(e != 0)
    def _():
        out_ref[rows, :] = out_ref[rows, :] + contrib


def _ffn_body(x_ref, f_ref, g2_ref, b2_ref, wf1_ref, bf1_ref, wf2_ref,
              bf2_ref, o_ref):
    x2 = x_ref[...] + f_ref[...]
    m = jnp.mean(x2, axis=1, keepdims=True)
    xc = x2 - m
    v = jnp.mean(xc * xc, axis=1, keepdims=True)
    xn = xc * lax.rsqrt(v + 1e-5) * g2_ref[...] + b2_ref[...]
    h = jnp.dot(xn, wf1_ref[...], preferred_element_type=F32) + bf1_ref[...]
    h = 0.5 * h * (1.0 + lax.erf(h / jnp.sqrt(2.0).astype(F32)))
    o_ref[...] = x2 + jnp.dot(h, wf2_ref[...],
                              preferred_element_type=F32) + bf2_ref[...]


def kernel(x, W_gate, g1, bn1, g2, bn2, Wf1, bf1, Wf2, bf2,
           W_in, conv_w, conv_b, W_xproj, W_dt, b_dt, A_log, Dp, W_out):
    B, L, D = x.shape
    E = W_gate.shape[1]
    d_inner = conv_w.shape[1]
    K = conv_w.shape[2]
    d_state = A_log.shape[2]
    dt_rank = W_dt.shape[1]
    C = 256
    NC = L // C
    x2d = x.reshape(L, D)
    cwT = jnp.swapaxes(conv_w, 1, 2)
    AT = jnp.swapaxes(A_log, 1, 2)
    row = lambda a: a.reshape(1, -1)

    xn, p = pl.pallas_call(
        _router_body,
        grid=(NC,),
        in_specs=[
            pl.BlockSpec((C, D), lambda c: (c, 0)),
            pl.BlockSpec((1, D), lambda c: (0, 0)),
            pl.BlockSpec((1, D), lambda c: (0, 0)),
            pl.BlockSpec((D, E), lambda c: (0, 0)),
        ],
        out_specs=[
            pl.BlockSpec((C, D), lambda c: (c, 0)),
            pl.BlockSpec((C, E), lambda c: (c, 0)),
        ],
        out_shape=[
            jax.ShapeDtypeStruct((L, D), F32),
            jax.ShapeDtypeStruct((L, E), F32),
        ],
    )(x2d, row(g1), row(bn1), W_gate)

    wT = _make_sc_router(E, L)(p.T)
    w = wT.T

    body = functools.partial(_mamba_body, C=C, d_inner=d_inner,
                             d_state=d_state, dt_rank=dt_rank, K=K)
    fused = pl.pallas_call(
        body,
        grid=(E, NC),
        in_specs=[
            pl.BlockSpec((C, D), lambda e, c: (c, 0)),
            pl.BlockSpec((C, E), lambda e, c: (c, 0)),
            pl.BlockSpec((1, D, 2 * d_inner), lambda e, c: (e, 0, 0)),
            pl.BlockSpec((1, K, d_inner), lambda e, c: (e, 0, 0)),
            pl.BlockSpec((1, 1, d_inner), lambda e, c: (e, 0, 0)),
            pl.BlockSpec((1, d_inner, dt_rank + 2 * d_state),
                         lambda e, c: (e, 0, 0)),
            pl.BlockSpec((1, dt_rank, d_inner), lambda e, c: (e, 0, 0)),
            pl.BlockSpec((1, 1, d_inner), lambda e, c: (e, 0, 0)),
            pl.BlockSpec((1, d_state, d_inner), lambda e, c: (e, 0, 0)),
            pl.BlockSpec((1, 1, d_inner), lambda e, c: (e, 0, 0)),
            pl.BlockSpec((1, d_inner, D), lambda e, c: (e, 0, 0)),
        ],
        out_specs=pl.BlockSpec((L, D), lambda e, c: (0, 0)),
        out_shape=jax.ShapeDtypeStruct((L, D), F32),
        scratch_shapes=[
            pltpu.VMEM((C + 8, d_inner), F32),
            pltpu.VMEM((d_state, d_inner), F32),
            pltpu.VMEM((C, d_inner), F32),
            pltpu.VMEM((C, d_inner), F32),
            pltpu.VMEM((C, d_inner), F32),
            pltpu.VMEM((C, d_inner), F32),
            pltpu.VMEM((C, 2 * d_state), F32),
        ],
        compiler_params=pltpu.CompilerParams(
            dimension_semantics=("arbitrary", "arbitrary")),
    )(xn, w, W_in, cwT, conv_b[:, None], W_xproj, W_dt, b_dt[:, None],
      AT, Dp[:, None], W_out)

    out = pl.pallas_call(
        _ffn_body,
        grid=(NC,),
        in_specs=[
            pl.BlockSpec((C, D), lambda c: (c, 0)),
            pl.BlockSpec((C, D), lambda c: (c, 0)),
            pl.BlockSpec((1, D), lambda c: (0, 0)),
            pl.BlockSpec((1, D), lambda c: (0, 0)),
            pl.BlockSpec((D, 2 * D), lambda c: (0, 0)),
            pl.BlockSpec((1, 2 * D), lambda c: (0, 0)),
            pl.BlockSpec((2 * D, D), lambda c: (0, 0)),
            pl.BlockSpec((1, D), lambda c: (0, 0)),
        ],
        out_specs=pl.BlockSpec((C, D), lambda c: (c, 0)),
        out_shape=jax.ShapeDtypeStruct((L, D), F32),
    )(x2d, fused, row(g2), row(bn2), Wf1, row(bf1), Wf2, row(bf2))

    return out.reshape(B, L, D)
